# Initial kernel scaffold; baseline (speedup 1.0000x reference)
import jax
import jax.numpy as jnp
from jax.experimental import pallas as pl


def kernel(x):
    raise NotImplementedError("write your pallas kernel here")


# speedup vs baseline: 14.8375x; 14.8375x over previous
import functools

import jax
import numpy as np
import jax.numpy as jnp
from jax import lax
from jax.experimental import pallas as pl
from jax.experimental.pallas import tpu as pltpu
from jax.experimental.pallas import tpu_sc as plsc

N = 64 * 32768
RANK = N // 2 - 1
L = 16
NC, NS = 2, 16
W = NC * NS
CHUNK = N // W
VECS = CHUNK // L
NB = 256
IMIN = np.int32(-(2 ** 31))
IMAX = np.int32(2 ** 31 - 1)


def _mesh():
    return plsc.VectorSubcoreMesh(core_axis_name="c", subcore_axis_name="s")


def _wid():
    return lax.axis_index("s") * NC + lax.axis_index("c")


def _full(v):
    return jnp.full((L,), v, jnp.int32)


def _key16(v_f32):
    b = lax.bitcast_convert_type(v_f32, jnp.int32)
    m = lax.shift_right_arithmetic(b, _full(31))
    return b ^ (m | IMIN)


def _merge_find(part_ref, r_rem):
    iota = lax.iota(jnp.int32, L)
    bin_ = np.int32(-1)
    below = np.int32(0)
    total = np.int32(0)
    for c in range(NB // L):
        def tacc(t, a, c=c):
            return a + part_ref[pl.ds(t * NB + c * L, L)]
        acc = lax.fori_loop(0, W, tacc, jnp.zeros((L,), jnp.int32))
        cs = plsc.cumsum(acc)
        mask = (total + cs) > r_rem
        found_here = jnp.any(mask)
        ffs = jnp.where(found_here, plsc.all_reduce_ffs(mask), np.int32(L))
        already = bin_ >= 0
        new_here = jnp.logical_and(found_here, jnp.logical_not(already))
        bin_ = jnp.where(new_here, np.int32(c * L) + ffs, bin_)
        below_here = jnp.sum(jnp.where(iota < ffs, acc, 0))
        below = below + jnp.where(already, np.int32(0), below_here)
        total = total + jnp.sum(acc)
    return bin_, below


def _chain(h_hbms, partv):
    prefix = np.int32(0)
    r_rem = np.int32(RANK)
    for h in h_hbms:
        pltpu.sync_copy(h, partv)
        b_, below = _merge_find(partv, r_rem)
        prefix = (prefix << 8) | b_
        r_rem = r_rem - below
    return prefix, r_rem


def _make_pass(p):
    nprev = p - 1
    shift = 32 - 8 * p
    scratch = [
        pltpu.VMEM((CHUNK,), jnp.float32),
        pltpu.VMEM((L * NB,), jnp.int32),
        pltpu.VMEM((NB,), jnp.int32),
    ]
    if nprev:
        scratch.append(pltpu.VMEM((W * NB,), jnp.int32))

    @functools.partial(
        pl.kernel,
        out_type=jax.ShapeDtypeStruct((W * NB,), jnp.int32),
        mesh=_mesh(),
        compiler_params=pltpu.CompilerParams(needs_layout_passes=False),
        scratch_types=scratch,
    )
    def body(*args):
        x_hbm = args[0]
        h_hbms = args[1:1 + nprev]
        if nprev:
            xv, histv, outv, partv = args[2 + nprev:]
        else:
            xv, histv, outv = args[2 + nprev:]
            partv = None
        out_hbm = args[1 + nprev]
        wid = _wid()
        pltpu.sync_copy(x_hbm.at[pl.ds(wid * CHUNK, CHUNK)], xv)
        prefix, _ = _chain(h_hbms, partv) if nprev else (np.int32(0), 0)

        def zbody(i, _):
            histv[pl.ds(i * L, L)] = jnp.zeros((L,), jnp.int32)
            return 0
        lax.fori_loop(0, (L * NB) // L, zbody, 0)

        base = lax.iota(jnp.int32, L) * NB
        ones = jnp.ones((L,), jnp.int32)
        shift_v = _full(shift)
        hi_v = _full(shift + 8)

        def hbody(i, _):
            key = _key16(xv[pl.ds(i * L, L)])
            dig = lax.shift_right_logical(key, shift_v) & 255
            if nprev:
                m = lax.shift_right_logical(key, hi_v) == prefix
                plsc.addupdate_scatter(histv, [base + dig], ones, mask=m)
            else:
                plsc.addupdate_scatter(histv, [base + dig], ones)
            return 0
        lax.fori_loop(0, VECS, hbody, 0)

        for c in range(NB // L):
            def lacc(l, a, c=c):
                return a + histv[pl.ds(l * NB + c * L, L)]
            outv[pl.ds(c * L, L)] = lax.fori_loop(
                0, L, lacc, jnp.zeros((L,), jnp.int32))
        pltpu.sync_copy(outv, out_hbm.at[pl.ds(wid * NB, NB)])

    return body


_pass1 = _make_pass(1)
_pass2 = _make_pass(2)
_pass3 = _make_pass(3)
_pass4 = _make_pass(4)


@functools.partial(
    pl.kernel,
    out_type=jax.ShapeDtypeStruct((W * L,), jnp.int32),
    mesh=_mesh(),
    compiler_params=pltpu.CompilerParams(needs_layout_passes=False),
    scratch_types=[
        pltpu.VMEM((CHUNK,), jnp.float32),
        pltpu.VMEM((W * NB,), jnp.int32),
        pltpu.VMEM((L,), jnp.int32),
    ],
)
def _pass5(x_hbm, h1, h2, h3, h4, out_hbm, xv, partv, outv):
    wid = _wid()
    pltpu.sync_copy(x_hbm.at[pl.ds(wid * CHUNK, CHUNK)], xv)
    key_a, _ = _chain((h1, h2, h3, h4), partv)
    keyc_a = key_a ^ IMIN

    def sbody(i, carry):
        cnt, mn = carry
        keyc = _key16(xv[pl.ds(i * L, L)]) ^ IMIN
        le = keyc <= keyc_a
        cnt = cnt + jnp.where(le, 1, 0)
        mn = jnp.where(le, mn, jnp.minimum(mn, keyc))
        return cnt, mn

    cnt, mn = lax.fori_loop(
        0, VECS, sbody,
        (jnp.zeros((L,), jnp.int32), jnp.full((L,), IMAX, jnp.int32)))
    iota = lax.iota(jnp.int32, L)
    res = jnp.where(iota == 0, jnp.sum(cnt),
                    jnp.where(iota == 1, jnp.min(mn), 0))
    outv[...] = res
    pltpu.sync_copy(outv, out_hbm.at[pl.ds(wid * L, L)])


@functools.partial(
    pl.kernel,
    out_type=jax.ShapeDtypeStruct((L,), jnp.float32),
    mesh=_mesh(),
    compiler_params=pltpu.CompilerParams(needs_layout_passes=False),
    scratch_types=[
        pltpu.VMEM((W * NB,), jnp.int32),
        pltpu.VMEM((W * L,), jnp.int32),
        pltpu.VMEM((L,), jnp.float32),
    ],
)
def _final(h1, h2, h3, h4, c_hbm, out_hbm, partv, cv, outv):
    wid = _wid()

    @pl.when(wid == 0)
    def _():
        key_a, _ = _chain((h1, h2, h3, h4), partv)
        pltpu.sync_copy(c_hbm, cv)
        iota = lax.iota(jnp.int32, L)

        def rbody(t, carry):
            c_acc, m_acc = carry
            chunk = cv[pl.ds(t * L, L)]
            c_acc = c_acc + jnp.where(iota == 0, chunk, 0)
            m_acc = jnp.minimum(m_acc, jnp.where(iota == 1, chunk, IMAX))
            return c_acc, m_acc

        c_acc, m_acc = lax.fori_loop(
            0, W, rbody,
            (jnp.zeros((L,), jnp.int32), jnp.full((L,), IMAX, jnp.int32)))
        cnt_le = jnp.sum(c_acc)
        mn_above = jnp.min(m_acc)
        key_b = jnp.where(cnt_le >= np.int32(RANK + 2),
                          key_a, mn_above ^ IMIN)
        keys = jnp.where(iota == 1, key_b, key_a)
        bits = jnp.where(keys < 0, keys ^ IMIN, ~keys)
        outv[...] = lax.bitcast_convert_type(bits, jnp.float32)
        pltpu.sync_copy(outv, out_hbm)


def kernel(x):
    flat = x.reshape(-1)
    h1 = _pass1(flat)
    h2 = _pass2(flat, h1)
    h3 = _pass3(flat, h1, h2)
    h4 = _pass4(flat, h1, h2, h3)
    c = _pass5(flat, h1, h2, h3, h4)
    out = _final(h1, h2, h3, h4, c)
    return out[0], out[1], out[2]


# speedup vs baseline: 16.1848x; 1.0908x over previous
import functools

import jax
import numpy as np
import jax.numpy as jnp
from jax import lax
from jax.experimental import pallas as pl
from jax.experimental.pallas import tpu as pltpu
from jax.experimental.pallas import tpu_sc as plsc

N = 64 * 32768
RANK = N // 2 - 1
L = 16
NC, NS = 2, 16
W = NC * NS
CHUNK = N // W
VECS = CHUNK // L
NB = 256
IMIN = np.int32(-(2 ** 31))
IMAX = np.int32(2 ** 31 - 1)


def _mesh():
    return plsc.VectorSubcoreMesh(core_axis_name="c", subcore_axis_name="s")


def _wid():
    return lax.axis_index("s") * NC + lax.axis_index("c")


def _full(v):
    return jnp.full((L,), v, jnp.int32)


def _key16(v_f32):
    b = lax.bitcast_convert_type(v_f32, jnp.int32)
    m = lax.shift_right_arithmetic(b, _full(31))
    return b ^ (m | IMIN)


def _merge_find(part_ref, r_rem):
    iota = lax.iota(jnp.int32, L)
    bin_ = np.int32(-1)
    below = np.int32(0)
    total = np.int32(0)
    for c in range(NB // L):
        def tacc(t, a, c=c):
            for u in range(4):
                a = a + part_ref[pl.ds((t * 4 + u) * NB + c * L, L)]
            return a
        acc = lax.fori_loop(0, W // 4, tacc, jnp.zeros((L,), jnp.int32))
        cs = plsc.cumsum(acc)
        mask = (total + cs) > r_rem
        found_here = jnp.any(mask)
        ffs = jnp.where(found_here, plsc.all_reduce_ffs(mask), np.int32(L))
        already = bin_ >= 0
        new_here = jnp.logical_and(found_here, jnp.logical_not(already))
        bin_ = jnp.where(new_here, np.int32(c * L) + ffs, bin_)
        below_here = jnp.sum(jnp.where(iota < ffs, acc, 0))
        below = below + jnp.where(already, np.int32(0), below_here)
        total = total + jnp.sum(acc)
    return bin_, below


def _chain(h_hbms, partv):
    prefix = np.int32(0)
    r_rem = np.int32(RANK)
    for h in h_hbms:
        pltpu.sync_copy(h, partv)
        b_, below = _merge_find(partv, r_rem)
        prefix = (prefix << 8) | b_
        r_rem = r_rem - below
    return prefix, r_rem


def _make_pass(p):
    nprev = p - 1
    shift = 32 - 8 * p
    scratch = [
        pltpu.VMEM((CHUNK,), jnp.float32),
        pltpu.VMEM((L * NB,), jnp.int32),
        pltpu.VMEM((NB,), jnp.int32),
        pltpu.SemaphoreType.DMA,
    ]
    if nprev:
        scratch.append(pltpu.VMEM((W * NB,), jnp.int32))

    @functools.partial(
        pl.kernel,
        out_type=jax.ShapeDtypeStruct((W * NB,), jnp.int32),
        mesh=_mesh(),
        compiler_params=pltpu.CompilerParams(needs_layout_passes=False),
        scratch_types=scratch,
    )
    def body(*args):
        x_hbm = args[0]
        h_hbms = args[1:1 + nprev]
        if nprev:
            xv, histv, outv, sem, partv = args[2 + nprev:]
        else:
            xv, histv, outv, sem = args[2 + nprev:]
            partv = None
        out_hbm = args[1 + nprev]
        wid = _wid()
        cp = pltpu.async_copy(x_hbm.at[pl.ds(wid * CHUNK, CHUNK)], xv, sem)

        zero = jnp.zeros((L,), jnp.int32)

        def zbody(i, _):
            for u in range(8):
                histv[pl.ds(i * (L * 8) + u * L, L)] = zero
            return 0
        lax.fori_loop(0, (L * NB) // (L * 8), zbody, 0)
        prefix, _ = _chain(h_hbms, partv) if nprev else (np.int32(0), 0)
        cp.wait()

        base = lax.iota(jnp.int32, L) * NB
        ones = jnp.ones((L,), jnp.int32)
        shift_v = _full(shift)
        hi_v = _full(shift + 8)

        def hbody(i, _):
            for u in range(8):
                key = _key16(xv[pl.ds(i * (L * 8) + u * L, L)])
                dig = lax.shift_right_logical(key, shift_v) & 255
                if nprev:
                    m = lax.shift_right_logical(key, hi_v) == prefix
                    plsc.addupdate_scatter(histv, [base + dig], ones, mask=m)
                else:
                    plsc.addupdate_scatter(histv, [base + dig], ones)
            return 0
        lax.fori_loop(0, VECS // 8, hbody, 0)

        for c in range(NB // L):
            acc = histv[pl.ds(c * L, L)]
            for l in range(1, L):
                acc = acc + histv[pl.ds(l * NB + c * L, L)]
            outv[pl.ds(c * L, L)] = acc
        pltpu.sync_copy(outv, out_hbm.at[pl.ds(wid * NB, NB)])

    return body


_pass1 = _make_pass(1)
_pass2 = _make_pass(2)
_pass3 = _make_pass(3)
_pass4 = _make_pass(4)


@functools.partial(
    pl.kernel,
    out_type=jax.ShapeDtypeStruct((W * L,), jnp.int32),
    mesh=_mesh(),
    compiler_params=pltpu.CompilerParams(needs_layout_passes=False),
    scratch_types=[
        pltpu.VMEM((CHUNK,), jnp.float32),
        pltpu.VMEM((W * NB,), jnp.int32),
        pltpu.VMEM((L,), jnp.int32),
        pltpu.SemaphoreType.DMA,
    ],
)
def _pass5(x_hbm, h1, h2, h3, h4, out_hbm, xv, partv, outv, sem):
    wid = _wid()
    cp = pltpu.async_copy(x_hbm.at[pl.ds(wid * CHUNK, CHUNK)], xv, sem)
    key_a, _ = _chain((h1, h2, h3, h4), partv)
    keyc_a = key_a ^ IMIN
    cp.wait()

    def sbody(i, carry):
        cnt, mn = carry
        for u in range(8):
            keyc = _key16(xv[pl.ds(i * (L * 8) + u * L, L)]) ^ IMIN
            le = keyc <= keyc_a
            cnt = cnt + jnp.where(le, 1, 0)
            mn = jnp.where(le, mn, jnp.minimum(mn, keyc))
        return cnt, mn

    cnt, mn = lax.fori_loop(
        0, VECS // 8, sbody,
        (jnp.zeros((L,), jnp.int32), jnp.full((L,), IMAX, jnp.int32)))
    iota = lax.iota(jnp.int32, L)
    res = jnp.where(iota == 0, jnp.sum(cnt),
                    jnp.where(iota == 1, jnp.min(mn), 0))
    outv[...] = res
    pltpu.sync_copy(outv, out_hbm.at[pl.ds(wid * L, L)])


@functools.partial(
    pl.kernel,
    out_type=jax.ShapeDtypeStruct((L,), jnp.float32),
    mesh=_mesh(),
    compiler_params=pltpu.CompilerParams(needs_layout_passes=False),
    scratch_types=[
        pltpu.VMEM((W * NB,), jnp.int32),
        pltpu.VMEM((W * L,), jnp.int32),
        pltpu.VMEM((L,), jnp.float32),
    ],
)
def _final(h1, h2, h3, h4, c_hbm, out_hbm, partv, cv, outv):
    wid = _wid()

    @pl.when(wid == 0)
    def _():
        key_a, _ = _chain((h1, h2, h3, h4), partv)
        pltpu.sync_copy(c_hbm, cv)
        iota = lax.iota(jnp.int32, L)

        def rbody(t, carry):
            c_acc, m_acc = carry
            chunk = cv[pl.ds(t * L, L)]
            c_acc = c_acc + jnp.where(iota == 0, chunk, 0)
            m_acc = jnp.minimum(m_acc, jnp.where(iota == 1, chunk, IMAX))
            return c_acc, m_acc

        c_acc, m_acc = lax.fori_loop(
            0, W, rbody,
            (jnp.zeros((L,), jnp.int32), jnp.full((L,), IMAX, jnp.int32)))
        cnt_le = jnp.sum(c_acc)
        mn_above = jnp.min(m_acc)
        key_b = jnp.where(cnt_le >= np.int32(RANK + 2),
                          key_a, mn_above ^ IMIN)
        keys = jnp.where(iota == 1, key_b, key_a)
        bits = jnp.where(keys < 0, keys ^ IMIN, ~keys)
        outv[...] = lax.bitcast_convert_type(bits, jnp.float32)
        pltpu.sync_copy(outv, out_hbm)


def kernel(x):
    flat = x.reshape(-1)
    h1 = _pass1(flat)
    h2 = _pass2(flat, h1)
    h3 = _pass3(flat, h1, h2)
    h4 = _pass4(flat, h1, h2, h3)
    c = _pass5(flat, h1, h2, h3, h4)
    out = _final(h1, h2, h3, h4, c)
    return out[0], out[1], out[2]


# speedup vs baseline: 26.6956x; 1.6494x over previous
import functools

import jax
import numpy as np
import jax.numpy as jnp
from jax import lax
from jax.experimental import pallas as pl
from jax.experimental.pallas import tpu as pltpu
from jax.experimental.pallas import tpu_sc as plsc

N = 64 * 32768
RANK = N // 2 - 1
L = 16
NC, NS = 2, 16
W = NC * NS
CHUNK = N // W
VECS = CHUNK // L
SHIFTS = (21, 10, 0)
NBS = (2048, 2048, 1024)
IMIN = np.int32(-(2 ** 31))
IMAX = np.int32(2 ** 31 - 1)


def _mesh():
    return plsc.VectorSubcoreMesh(core_axis_name="c", subcore_axis_name="s")


def _full(v):
    return jnp.full((L,), v, jnp.int32)


def _key16(v_f32):
    b = lax.bitcast_convert_type(v_f32, jnp.int32)
    m = lax.shift_right_arithmetic(b, _full(31))
    return b ^ (m | IMIN)


def _merge_find(partv, nb, r_rem):
    iota = lax.iota(jnp.int32, L)

    def body(c, carry):
        bin_, below, total, ceq = carry
        acc = partv[pl.ds(c * L, L)] + partv[pl.ds(nb + c * L, L)]
        cs = plsc.cumsum(acc)
        mask = (total + cs) > r_rem
        found_here = jnp.any(mask)
        ffs = jnp.where(found_here, plsc.all_reduce_ffs(mask), np.int32(L))
        already = bin_ >= 0
        new_here = jnp.logical_and(found_here, jnp.logical_not(already))
        bin_ = jnp.where(new_here, c * L + ffs, bin_)
        below_here = jnp.sum(jnp.where(iota < ffs, acc, 0))
        eq_here = jnp.sum(jnp.where(iota == ffs, acc, 0))
        upd = jnp.logical_not(already)
        below = jnp.where(upd, below + below_here, below)
        ceq = jnp.where(jnp.logical_and(upd, found_here), eq_here, ceq)
        total = total + jnp.sum(acc)
        return bin_, below, total, ceq

    bin_, below, _, ceq = lax.fori_loop(
        0, nb // L, body, (_full(-1), _full(0), _full(0), _full(0)))
    return bin_, below, ceq


def _chain(h_hbms, partv, upto):
    prefix = _full(0)
    r_rem = _full(RANK)
    ceq = _full(0)
    for p in range(upto):
        nb = NBS[p]
        pltpu.sync_copy(h_hbms[p].at[pl.ds(0, 2 * nb)],
                        partv.at[pl.ds(0, 2 * nb)])
        b_, below, ceq = _merge_find(partv, nb, r_rem)
        bits = int(np.log2(nb))
        prefix = (prefix << bits) | b_
        r_rem = r_rem - below
    return prefix, r_rem, ceq


def _make_pass(p):
    nb = NBS[p]
    shift = SHIFTS[p]
    out_words = 2 * nb + (W * L if p == 2 else 0)
    scratch = [
        pltpu.VMEM((CHUNK,), jnp.float32),
        pltpu.VMEM((L * nb,), jnp.int32),
        pltpu.VMEM((nb,), jnp.int32),
        pltpu.VMEM((nb,), jnp.int32),
        pltpu.VMEM_SHARED((nb,), jnp.int32),
        pltpu.SemaphoreType.DMA,
    ]
    if p:
        scratch.append(pltpu.VMEM((2 * NBS[0],), jnp.int32))

    @functools.partial(
        pl.kernel,
        out_type=jax.ShapeDtypeStruct((out_words,), jnp.int32),
        mesh=_mesh(),
        compiler_params=pltpu.CompilerParams(needs_layout_passes=False),
        scratch_types=scratch,
    )
    def body(*args):
        x_hbm = args[0]
        h_hbms = args[1:1 + p]
        out_hbm = args[1 + p]
        if p:
            xv, histv, outv, idxv, shared, sem, partv = args[2 + p:]
        else:
            xv, histv, outv, idxv, shared, sem = args[2 + p:]
            partv = None
        sid = lax.axis_index("s")
        core = lax.axis_index("c")
        wid = sid * NC + core
        cp = pltpu.async_copy(x_hbm.at[pl.ds(wid * CHUNK, CHUNK)], xv, sem)

        iota = lax.iota(jnp.int32, L)
        zero = jnp.zeros((L,), jnp.int32)

        def zobody(c, _):
            outv[pl.ds(c * L, L)] = zero
            idxv[pl.ds(c * L, L)] = iota + c * L
            return 0
        lax.fori_loop(0, nb // L, zobody, 0)

        @pl.when(sid == 0)
        def _():
            pltpu.sync_copy(outv, shared)
        plsc.subcore_barrier()

        def zbody(i, _):
            for u in range(8):
                histv[pl.ds(i * (L * 8) + u * L, L)] = zero
            return 0
        lax.fori_loop(0, (L * nb) // (L * 8), zbody, 0)

        if p:
            prefix, _, _ = _chain(h_hbms, partv, p)
        cp.wait()

        base = iota * nb
        ones = jnp.ones((L,), jnp.int32)
        shift_v = _full(shift)
        hi_v = _full(shift + int(np.log2(nb)))
        mask_dig = _full(nb - 1)

        def hbody(i, mn):
            for u in range(8):
                key = _key16(xv[pl.ds(i * (L * 8) + u * L, L)])
                dig = lax.shift_right_logical(key, shift_v) & mask_dig
                if p:
                    hi = lax.shift_right_logical(key, hi_v)
                    m = hi == prefix
                    plsc.addupdate_scatter(histv, [base + dig], ones, mask=m)
                    if p == 2:
                        keyc = key ^ IMIN
                        mn = jnp.where(hi > prefix,
                                       jnp.minimum(mn, keyc), mn)
                else:
                    plsc.addupdate_scatter(histv, [base + dig], ones)
            return mn

        mn = lax.fori_loop(0, VECS // 8, hbody, jnp.full((L,), IMAX, jnp.int32))

        def rbody(c, _):
            acc = histv[pl.ds(c * L, L)]
            for l in range(1, L):
                acc = acc + histv[pl.ds(l * nb + c * L, L)]
            outv[pl.ds(c * L, L)] = acc
            return 0
        lax.fori_loop(0, nb // L, rbody, 0)

        pltpu.sync_copy(outv, shared.at[idxv], add=True)
        plsc.subcore_barrier()

        @pl.when(sid == 0)
        def _():
            pltpu.sync_copy(shared, out_hbm.at[pl.ds(core * nb, nb)])

        if p == 2:
            outv[pl.ds(0, L)] = mn
            pltpu.sync_copy(
                outv.at[pl.ds(0, L)],
                out_hbm.at[pl.ds(2 * nb + wid * L, L)])

    return body


_pass1 = _make_pass(0)
_pass2 = _make_pass(1)
_pass3 = _make_pass(2)


@functools.partial(
    pl.kernel,
    out_type=jax.ShapeDtypeStruct((L,), jnp.float32),
    mesh=_mesh(),
    compiler_params=pltpu.CompilerParams(needs_layout_passes=False),
    scratch_types=[
        pltpu.VMEM((2 * NBS[0],), jnp.int32),
        pltpu.VMEM((W * L,), jnp.int32),
        pltpu.VMEM((L,), jnp.float32),
    ],
)
def _final(h1, h2, h3m, out_hbm, partv, cv, outv):
    sid = lax.axis_index("s")
    core = lax.axis_index("c")
    iota = lax.iota(jnp.int32, L)
    nb3 = NBS[2]

    @pl.when(jnp.logical_and(sid == 0, core == 0))
    def _():
        key_a, r_rem, ceq = _chain((h1, h2, h3m), partv, 3)
        bin3 = key_a & (nb3 - 1)

        def nzbody(c, nxt):
            acc = partv[pl.ds(c * L, L)] + partv[pl.ds(nb3 + c * L, L)]
            gi = c * L + iota
            cand = jnp.where(jnp.logical_and(acc > 0, gi > bin3),
                             gi, _full(nb3))
            return jnp.minimum(nxt, cand)

        nxt = lax.fori_loop(0, nb3 // L, nzbody, _full(nb3))
        nxt_bin = jnp.min(nxt)
        keyc_cand = jnp.where(
            nxt_bin < nb3,
            (((key_a >> 10) << 10) | nxt_bin) ^ IMIN, IMAX)

        pltpu.sync_copy(h3m.at[pl.ds(2 * nb3, W * L)], cv)

        def mbody(t, m_acc):
            return jnp.minimum(m_acc, cv[pl.ds(t * L, L)])

        mn_next = jnp.min(lax.fori_loop(
            0, W, mbody, jnp.full((L,), IMAX, jnp.int32)))

        cnt_le = (_full(RANK) - r_rem) + ceq
        keyc_b = jnp.where(cnt_le >= np.int32(RANK + 2),
                           key_a ^ IMIN,
                           jnp.minimum(keyc_cand, mn_next))
        key_b = keyc_b ^ IMIN
        keys = jnp.where(iota == 1, key_b, key_a)
        bits = jnp.where(keys < 0, keys ^ IMIN, ~keys)
        outv[...] = lax.bitcast_convert_type(bits, jnp.float32)
        pltpu.sync_copy(outv, out_hbm)


def kernel(x):
    flat = x.reshape(-1)
    h1 = _pass1(flat)
    h2 = _pass2(flat, h1)
    h3 = _pass3(flat, h1, h2)
    out = _final(h1, h2, h3)
    return out[0], out[1], out[2]


# speedup vs baseline: 30.4500x; 1.1406x over previous
import functools

import jax
import numpy as np
import jax.numpy as jnp
from jax import lax
from jax.experimental import pallas as pl
from jax.experimental.pallas import tpu as pltpu
from jax.experimental.pallas import tpu_sc as plsc

ROWS, COLS = 64, 32768
N = ROWS * COLS
RANK = N // 2 - 1
L = 16
NC, NS = 2, 16
W = NC * NS
RPW = ROWS // W
CHUNK = N // W
VECS = CHUNK // L
SHIFTS = (21, 10, 0)
NBS = (2048, 2048, 1024)
IMIN = np.int32(-(2 ** 31))
IMAX = np.int32(2 ** 31 - 1)


def _mesh():
    return plsc.VectorSubcoreMesh(core_axis_name="c", subcore_axis_name="s")


def _full(v):
    return jnp.full((L,), v, jnp.int32)


def _key16(v_f32):
    b = lax.bitcast_convert_type(v_f32, jnp.int32)
    m = lax.shift_right_arithmetic(b, _full(31))
    return b ^ (m | IMIN)


def _merge_find(partv, nb, r_rem):
    iota = lax.iota(jnp.int32, L)

    def body(c, carry):
        bin_, below, total, ceq = carry
        acc = partv[pl.ds(c * L, L)] + partv[pl.ds(nb + c * L, L)]
        cs = plsc.cumsum(acc)
        mask = (total + cs) > r_rem
        found_here = jnp.any(mask)
        ffs = jnp.where(found_here, plsc.all_reduce_ffs(mask), np.int32(L))
        already = bin_ >= 0
        new_here = jnp.logical_and(found_here, jnp.logical_not(already))
        bin_ = jnp.where(new_here, c * L + ffs, bin_)
        below_here = jnp.sum(jnp.where(iota < ffs, acc, 0))
        eq_here = jnp.sum(jnp.where(iota == ffs, acc, 0))
        upd = jnp.logical_not(already)
        below = jnp.where(upd, below + below_here, below)
        ceq = jnp.where(jnp.logical_and(upd, found_here), eq_here, ceq)
        total = total + jnp.sum(acc)
        return bin_, below, total, ceq

    bin_, below, _, ceq = lax.fori_loop(
        0, nb // L, body, (_full(-1), _full(0), _full(0), _full(0)))
    return bin_, below, ceq


def _chain(h_hbms, partv, upto):
    prefix = _full(0)
    r_rem = _full(RANK)
    ceq = _full(0)
    for p in range(upto):
        nb = NBS[p]
        pltpu.sync_copy(h_hbms[p].at[pl.ds(0, 2 * nb)],
                        partv.at[pl.ds(0, 2 * nb)])
        b_, below, ceq = _merge_find(partv, nb, r_rem)
        bits = int(np.log2(nb))
        prefix = (prefix << bits) | b_
        r_rem = r_rem - below
    return prefix, r_rem, ceq


def _make_pass(p):
    nb = NBS[p]
    shift = SHIFTS[p]
    out_words = 2 * nb + (W * L if p == 2 else 0)
    scratch = [
        pltpu.VMEM((CHUNK,), jnp.float32),
        pltpu.VMEM((nb,), jnp.int32),
        pltpu.VMEM((nb,), jnp.int32),
        pltpu.VMEM_SHARED((nb,), jnp.int32),
        pltpu.SemaphoreType.DMA,
    ]
    if p:
        scratch.append(pltpu.VMEM((2 * NBS[0],), jnp.int32))

    @functools.partial(
        pl.kernel,
        out_type=jax.ShapeDtypeStruct((out_words,), jnp.int32),
        mesh=_mesh(),
        compiler_params=pltpu.CompilerParams(needs_layout_passes=False),
        scratch_types=scratch,
    )
    def body(*args):
        x_hbm = args[0]
        h_hbms = args[1:1 + p]
        out_hbm = args[1 + p]
        if p:
            xv, histv, idxv, shared, sem, partv = args[2 + p:]
        else:
            xv, histv, idxv, shared, sem = args[2 + p:]
            partv = None
        sid = lax.axis_index("s")
        core = lax.axis_index("c")
        wid = sid * NC + core
        cps = [
            pltpu.async_copy(
                x_hbm.at[wid * RPW + r], xv.at[pl.ds(r * COLS, COLS)], sem)
            for r in range(RPW)
        ]

        iota = lax.iota(jnp.int32, L)
        zero = jnp.zeros((L,), jnp.int32)

        def zobody(c, _):
            histv[pl.ds(c * L, L)] = zero
            idxv[pl.ds(c * L, L)] = iota + c * L
            return 0
        lax.fori_loop(0, nb // L, zobody, 0)

        @pl.when(sid == 0)
        def _():
            pltpu.sync_copy(histv, shared)
        plsc.subcore_barrier()

        if p:
            prefix, _, _ = _chain(h_hbms, partv, p)
        for cp in cps:
            cp.wait()

        ones = jnp.ones((L,), jnp.int32)
        shift_v = _full(shift)
        hi_v = _full(shift + int(np.log2(nb)))
        mask_dig = _full(nb - 1)

        def hbody(i, mn):
            for u in range(8):
                key = _key16(xv[pl.ds(i * (L * 8) + u * L, L)])
                dig = lax.shift_right_logical(key, shift_v) & mask_dig
                if p:
                    hi = lax.shift_right_logical(key, hi_v)
                    m = hi == prefix
                    plsc.addupdate_scatter(histv, [dig], ones, mask=m)
                    if p == 2:
                        keyc = key ^ IMIN
                        mn = jnp.where(hi > prefix,
                                       jnp.minimum(mn, keyc), mn)
                else:
                    plsc.addupdate_scatter(histv, [dig], ones)
            return mn

        mn = lax.fori_loop(0, VECS // 8, hbody, jnp.full((L,), IMAX, jnp.int32))

        pltpu.sync_copy(histv, shared.at[idxv], add=True)
        plsc.subcore_barrier()

        @pl.when(sid == 0)
        def _():
            pltpu.sync_copy(shared, out_hbm.at[pl.ds(core * nb, nb)])

        if p == 2:
            idxv[pl.ds(0, L)] = mn
            pltpu.sync_copy(
                idxv.at[pl.ds(0, L)],
                out_hbm.at[pl.ds(2 * nb + wid * L, L)])

    return body


_pass1 = _make_pass(0)
_pass2 = _make_pass(1)
_pass3 = _make_pass(2)


@functools.partial(
    pl.kernel,
    out_type=jax.ShapeDtypeStruct((L,), jnp.float32),
    mesh=_mesh(),
    compiler_params=pltpu.CompilerParams(needs_layout_passes=False),
    scratch_types=[
        pltpu.VMEM((2 * NBS[0],), jnp.int32),
        pltpu.VMEM((W * L,), jnp.int32),
        pltpu.VMEM((L,), jnp.float32),
    ],
)
def _final(h1, h2, h3m, out_hbm, partv, cv, outv):
    sid = lax.axis_index("s")
    core = lax.axis_index("c")
    iota = lax.iota(jnp.int32, L)
    nb3 = NBS[2]

    @pl.when(jnp.logical_and(sid == 0, core == 0))
    def _():
        key_a, r_rem, ceq = _chain((h1, h2, h3m), partv, 3)
        bin3 = key_a & (nb3 - 1)

        def nzbody(c, nxt):
            acc = partv[pl.ds(c * L, L)] + partv[pl.ds(nb3 + c * L, L)]
            gi = c * L + iota
            cand = jnp.where(jnp.logical_and(acc > 0, gi > bin3),
                             gi, _full(nb3))
            return jnp.minimum(nxt, cand)

        nxt = lax.fori_loop(0, nb3 // L, nzbody, _full(nb3))
        nxt_bin = jnp.min(nxt)
        keyc_cand = jnp.where(
            nxt_bin < nb3,
            (((key_a >> 10) << 10) | nxt_bin) ^ IMIN, IMAX)

        pltpu.sync_copy(h3m.at[pl.ds(2 * nb3, W * L)], cv)

        def mbody(t, m_acc):
            return jnp.minimum(m_acc, cv[pl.ds(t * L, L)])

        mn_next = jnp.min(lax.fori_loop(
            0, W, mbody, jnp.full((L,), IMAX, jnp.int32)))

        cnt_le = (_full(RANK) - r_rem) + ceq
        keyc_b = jnp.where(cnt_le >= np.int32(RANK + 2),
                           key_a ^ IMIN,
                           jnp.minimum(keyc_cand, mn_next))
        key_b = keyc_b ^ IMIN
        keys = jnp.where(iota == 1, key_b, key_a)
        bits = jnp.where(keys < 0, keys ^ IMIN, ~keys)
        outv[...] = lax.bitcast_convert_type(bits, jnp.float32)
        pltpu.sync_copy(outv, out_hbm)


def kernel(x):
    h1 = _pass1(x)
    h2 = _pass2(x, h1)
    h3 = _pass3(x, h1, h2)
    out = _final(h1, h2, h3)
    return out[0], out[1], out[2]


# speedup vs baseline: 67.7637x; 2.2254x over previous
import functools

import jax
import numpy as np
import jax.numpy as jnp
from jax import lax
from jax.experimental import pallas as pl
from jax.experimental.pallas import tpu as pltpu
from jax.experimental.pallas import tpu_sc as plsc

ROWS, COLS = 64, 32768
N = ROWS * COLS
RANK = N // 2 - 1
L = 16
NC, NS = 2, 16
W = NC * NS
RPW = ROWS // W
CHUNK = N // W
VECS = CHUNK // L
SHIFTS = (21, 10, 0)
NBS = (2048, 2048, 1024)
IMIN = np.int32(-(2 ** 31))
IMAX = np.int32(2 ** 31 - 1)


def _mesh():
    return plsc.VectorSubcoreMesh(core_axis_name="c", subcore_axis_name="s")


def _full(v):
    return jnp.full((L,), v, jnp.int32)


def _key16(v_f32):
    b = lax.bitcast_convert_type(v_f32, jnp.int32)
    m = lax.shift_right_arithmetic(b, _full(31))
    return b ^ (m | IMIN)


def _merge_find(partv, nb, r_rem):
    iota = lax.iota(jnp.int32, L)

    def body(c, carry):
        bin_, below, total, ceq = carry
        acc = partv[pl.ds(c * L, L)] + partv[pl.ds(nb + c * L, L)]
        cs = plsc.cumsum(acc)
        mask = (total + cs) > r_rem
        found_here = jnp.any(mask)
        ffs = jnp.where(found_here, plsc.all_reduce_ffs(mask), np.int32(L))
        already = bin_ >= 0
        new_here = jnp.logical_and(found_here, jnp.logical_not(already))
        bin_ = jnp.where(new_here, c * L + ffs, bin_)
        below_here = jnp.sum(jnp.where(iota < ffs, acc, 0))
        eq_here = jnp.sum(jnp.where(iota == ffs, acc, 0))
        upd = jnp.logical_not(already)
        below = jnp.where(upd, below + below_here, below)
        ceq = jnp.where(jnp.logical_and(upd, found_here), eq_here, ceq)
        total = total + jnp.sum(acc)
        return bin_, below, total, ceq

    bin_, below, _, ceq = lax.fori_loop(
        0, nb // L, body, (_full(-1), _full(0), _full(0), _full(0)))
    return bin_, below, ceq


def _chain(h_hbms, partv, upto):
    prefix = _full(0)
    r_rem = _full(RANK)
    ceq = _full(0)
    for p in range(upto):
        nb = NBS[p]
        pltpu.sync_copy(h_hbms[p].at[pl.ds(0, 2 * nb)],
                        partv.at[pl.ds(0, 2 * nb)])
        b_, below, ceq = _merge_find(partv, nb, r_rem)
        bits = int(np.log2(nb))
        prefix = (prefix << bits) | b_
        r_rem = r_rem - below
    return prefix, r_rem, ceq


def _make_pass(p):
    nb = NBS[p]
    shift = SHIFTS[p]
    out_words = 2 * nb + (W * L if p == 2 else 0)
    scratch = [
        pltpu.VMEM((CHUNK,), jnp.float32),
        pltpu.VMEM((nb,), jnp.int32),
        pltpu.VMEM((nb,), jnp.int32),
        pltpu.VMEM_SHARED((nb,), jnp.int32),
        pltpu.SemaphoreType.DMA,
    ]
    if p:
        scratch.append(pltpu.VMEM((2 * NBS[0],), jnp.int32))

    @functools.partial(
        pl.kernel,
        out_type=jax.ShapeDtypeStruct((out_words,), jnp.int32),
        mesh=_mesh(),
        compiler_params=pltpu.CompilerParams(needs_layout_passes=False),
        scratch_types=scratch,
    )
    def body(*args):
        x_hbm = args[0]
        h_hbms = args[1:1 + p]
        out_hbm = args[1 + p]
        if p:
            xv, histv, idxv, shared, sem, partv = args[2 + p:]
        else:
            xv, histv, idxv, shared, sem = args[2 + p:]
            partv = None
        sid = lax.axis_index("s")
        core = lax.axis_index("c")
        wid = sid * NC + core
        cps = [
            pltpu.async_copy(
                x_hbm.at[wid * RPW + r], xv.at[pl.ds(r * COLS, COLS)], sem)
            for r in range(RPW)
        ]

        iota = lax.iota(jnp.int32, L)
        zero = jnp.zeros((L,), jnp.int32)

        def zobody(c, _):
            histv[pl.ds(c * L, L)] = zero
            idxv[pl.ds(c * L, L)] = iota + c * L
            return 0
        lax.fori_loop(0, nb // L, zobody, 0)

        @pl.when(sid == 0)
        def _():
            pltpu.sync_copy(histv, shared)
        plsc.subcore_barrier()

        if p:
            prefix, _, _ = _chain(h_hbms, partv, p)
        for cp in cps:
            cp.wait()

        ones = jnp.ones((L,), jnp.int32)
        shift_v = _full(shift)
        hi_v = _full(shift + int(np.log2(nb)))
        mask_dig = _full(nb - 1)

        @plsc.parallel_loop(0, VECS, carry=jnp.full((L,), IMAX, jnp.int32),
                            unroll=8)
        def mn(i, mn):
            key = _key16(xv[pl.ds(i * L, L)])
            dig = lax.shift_right_logical(key, shift_v) & mask_dig
            if p:
                hi = lax.shift_right_logical(key, hi_v)
                m = hi == prefix
                plsc.addupdate_scatter(histv, [dig], ones, mask=m)
                if p == 2:
                    mn = jnp.where(hi > prefix,
                                   jnp.minimum(mn, key ^ IMIN), mn)
            else:
                plsc.addupdate_scatter(histv, [dig], ones)
            return mn

        pltpu.sync_copy(histv, shared.at[idxv], add=True)
        plsc.subcore_barrier()

        @pl.when(sid == 0)
        def _():
            pltpu.sync_copy(shared, out_hbm.at[pl.ds(core * nb, nb)])

        if p == 2:
            idxv[pl.ds(0, L)] = mn
            pltpu.sync_copy(
                idxv.at[pl.ds(0, L)],
                out_hbm.at[pl.ds(2 * nb + wid * L, L)])

    return body


_pass1 = _make_pass(0)
_pass2 = _make_pass(1)
_pass3 = _make_pass(2)


@functools.partial(
    pl.kernel,
    out_type=jax.ShapeDtypeStruct((L,), jnp.float32),
    mesh=_mesh(),
    compiler_params=pltpu.CompilerParams(needs_layout_passes=False),
    scratch_types=[
        pltpu.VMEM((2 * NBS[0],), jnp.int32),
        pltpu.VMEM((W * L,), jnp.int32),
        pltpu.VMEM((L,), jnp.float32),
    ],
)
def _final(h1, h2, h3m, out_hbm, partv, cv, outv):
    sid = lax.axis_index("s")
    core = lax.axis_index("c")
    iota = lax.iota(jnp.int32, L)
    nb3 = NBS[2]

    @pl.when(jnp.logical_and(sid == 0, core == 0))
    def _():
        key_a, r_rem, ceq = _chain((h1, h2, h3m), partv, 3)
        bin3 = key_a & (nb3 - 1)

        def nzbody(c, nxt):
            acc = partv[pl.ds(c * L, L)] + partv[pl.ds(nb3 + c * L, L)]
            gi = c * L + iota
            cand = jnp.where(jnp.logical_and(acc > 0, gi > bin3),
                             gi, _full(nb3))
            return jnp.minimum(nxt, cand)

        nxt = lax.fori_loop(0, nb3 // L, nzbody, _full(nb3))
        nxt_bin = jnp.min(nxt)
        keyc_cand = jnp.where(
            nxt_bin < nb3,
            (((key_a >> 10) << 10) | nxt_bin) ^ IMIN, IMAX)

        pltpu.sync_copy(h3m.at[pl.ds(2 * nb3, W * L)], cv)

        def mbody(t, m_acc):
            return jnp.minimum(m_acc, cv[pl.ds(t * L, L)])

        mn_next = jnp.min(lax.fori_loop(
            0, W, mbody, jnp.full((L,), IMAX, jnp.int32)))

        cnt_le = (_full(RANK) - r_rem) + ceq
        keyc_b = jnp.where(cnt_le >= np.int32(RANK + 2),
                           key_a ^ IMIN,
                           jnp.minimum(keyc_cand, mn_next))
        key_b = keyc_b ^ IMIN
        keys = jnp.where(iota == 1, key_b, key_a)
        bits = jnp.where(keys < 0, keys ^ IMIN, ~keys)
        outv[...] = lax.bitcast_convert_type(bits, jnp.float32)
        pltpu.sync_copy(outv, out_hbm)


def kernel(x):
    h1 = _pass1(x)
    h2 = _pass2(x, h1)
    h3 = _pass3(x, h1, h2)
    out = _final(h1, h2, h3)
    return out[0], out[1], out[2]


# speedup vs baseline: 71.0774x; 1.0489x over previous
import functools

import jax
import numpy as np
import jax.numpy as jnp
from jax import lax
from jax.experimental import pallas as pl
from jax.experimental.pallas import tpu as pltpu
from jax.experimental.pallas import tpu_sc as plsc

ROWS, COLS = 64, 32768
N = ROWS * COLS
RANK = N // 2 - 1
L = 16
NC, NS = 2, 16
W = NC * NS
RPW = ROWS // W
CHUNK = N // W
VECS = CHUNK // L
SHIFTS = (21, 10, 0)
NBS = (2048, 2048, 1024)
IMIN = np.int32(-(2 ** 31))
IMAX = np.int32(2 ** 31 - 1)


def _mesh():
    return plsc.VectorSubcoreMesh(core_axis_name="c", subcore_axis_name="s")


def _full(v):
    return jnp.full((L,), v, jnp.int32)


def _key16(v_f32):
    b = lax.bitcast_convert_type(v_f32, jnp.int32)
    m = lax.shift_right_arithmetic(b, _full(31))
    return b ^ (m | IMIN)


def _merge_find(partv, nb, r_rem):
    iota = lax.iota(jnp.int32, L)

    def body(c, carry):
        bin_, below, total, ceq = carry
        acc = partv[pl.ds(c * L, L)] + partv[pl.ds(nb + c * L, L)]
        cs = plsc.cumsum(acc)
        mask = (total + cs) > r_rem
        found_here = jnp.any(mask)
        ffs = jnp.where(found_here, plsc.all_reduce_ffs(mask), np.int32(L))
        already = bin_ >= 0
        new_here = jnp.logical_and(found_here, jnp.logical_not(already))
        bin_ = jnp.where(new_here, c * L + ffs, bin_)
        below_here = jnp.sum(jnp.where(iota < ffs, acc, 0))
        eq_here = jnp.sum(jnp.where(iota == ffs, acc, 0))
        upd = jnp.logical_not(already)
        below = jnp.where(upd, below + below_here, below)
        ceq = jnp.where(jnp.logical_and(upd, found_here), eq_here, ceq)
        total = total + jnp.sum(acc)
        return bin_, below, total, ceq

    bin_, below, _, ceq = lax.fori_loop(
        0, nb // L, body, (_full(-1), _full(0), _full(0), _full(0)))
    return bin_, below, ceq


def _step(h_hbm, partv, p, prefix, r_rem):
    nb = NBS[p]
    pltpu.sync_copy(h_hbm.at[pl.ds(0, 2 * nb)], partv.at[pl.ds(0, 2 * nb)])
    b_, below, ceq = _merge_find(partv, nb, r_rem)
    prefix = (prefix << int(np.log2(nb))) | b_
    return prefix, r_rem - below, ceq


def _make_pass(p):
    nb = NBS[p]
    shift = SHIFTS[p]
    out_words = 2 * nb + (W * L if p == 2 else 0) + (2 * L if p else 0)
    state_off = 2 * nb + (W * L if p == 2 else 0)
    scratch = [
        pltpu.VMEM((CHUNK,), jnp.float32),
        pltpu.VMEM((nb,), jnp.int32),
        pltpu.VMEM((nb,), jnp.int32),
        pltpu.VMEM_SHARED((nb,), jnp.int32),
        pltpu.SemaphoreType.DMA,
    ]
    if p:
        scratch.append(pltpu.VMEM((2 * NBS[p - 1],), jnp.int32))

    @functools.partial(
        pl.kernel,
        out_type=jax.ShapeDtypeStruct((out_words,), jnp.int32),
        mesh=_mesh(),
        compiler_params=pltpu.CompilerParams(needs_layout_passes=False),
        scratch_types=scratch,
    )
    def body(*args):
        x_hbm = args[0]
        if p:
            h_prev = args[1]
            out_hbm = args[2]
            xv, histv, idxv, shared, sem, partv = args[3:]
        else:
            out_hbm = args[1]
            xv, histv, idxv, shared, sem = args[2:]
            partv = None
        sid = lax.axis_index("s")
        core = lax.axis_index("c")
        wid = sid * NC + core
        cps = [
            pltpu.async_copy(
                x_hbm.at[wid * RPW + r], xv.at[pl.ds(r * COLS, COLS)], sem)
            for r in range(RPW)
        ]

        iota = lax.iota(jnp.int32, L)
        zero = jnp.zeros((L,), jnp.int32)

        @plsc.parallel_loop(0, nb // L, unroll=8)
        def _(c):
            histv[pl.ds(c * L, L)] = zero
            idxv[pl.ds(c * L, L)] = iota + c * L

        @pl.when(sid == 0)
        def _():
            pltpu.sync_copy(histv, shared)
        plsc.subcore_barrier()

        if p == 1:
            prefix, r_rem, _ = _step(h_prev, partv, 0, _full(0), _full(RANK))
        elif p == 2:
            poff = 2 * NBS[p - 1]
            pltpu.sync_copy(h_prev.at[pl.ds(poff, 2 * L)],
                            partv.at[pl.ds(0, 2 * L)])
            prev_prefix = partv[pl.ds(0, L)]
            prev_r = partv[pl.ds(L, L)]
            prefix, r_rem, _ = _step(h_prev, partv, 1, prev_prefix, prev_r)
        if p:
            @pl.when(jnp.logical_and(sid == 0, core == 0))
            def _():
                partv[pl.ds(0, L)] = prefix
                partv[pl.ds(L, L)] = r_rem
                pltpu.sync_copy(partv.at[pl.ds(0, 2 * L)],
                                out_hbm.at[pl.ds(state_off, 2 * L)])
        for cp in cps:
            cp.wait()

        ones = jnp.ones((L,), jnp.int32)
        shift_v = _full(shift)
        hi_v = _full(shift + int(np.log2(nb)))
        mask_dig = _full(nb - 1)

        @plsc.parallel_loop(0, VECS, carry=jnp.full((L,), IMAX, jnp.int32),
                            unroll=8)
        def mn(i, mn):
            key = _key16(xv[pl.ds(i * L, L)])
            dig = lax.shift_right_logical(key, shift_v) & mask_dig
            if p:
                hi = lax.shift_right_logical(key, hi_v)
                m = hi == prefix
                plsc.addupdate_scatter(histv, [dig], ones, mask=m)
                if p == 2:
                    mn = jnp.where(hi > prefix,
                                   jnp.minimum(mn, key ^ IMIN), mn)
            else:
                plsc.addupdate_scatter(histv, [dig], ones)
            return mn

        pltpu.sync_copy(histv, shared.at[idxv], add=True)
        plsc.subcore_barrier()

        @pl.when(sid == 0)
        def _():
            pltpu.sync_copy(shared, out_hbm.at[pl.ds(core * nb, nb)])

        if p == 2:
            idxv[pl.ds(0, L)] = mn
            pltpu.sync_copy(
                idxv.at[pl.ds(0, L)],
                out_hbm.at[pl.ds(2 * nb + wid * L, L)])

    return body


_pass1 = _make_pass(0)
_pass2 = _make_pass(1)
_pass3 = _make_pass(2)


@functools.partial(
    pl.kernel,
    out_type=jax.ShapeDtypeStruct((L,), jnp.float32),
    mesh=_mesh(),
    compiler_params=pltpu.CompilerParams(needs_layout_passes=False),
    scratch_types=[
        pltpu.VMEM((2 * NBS[2],), jnp.int32),
        pltpu.VMEM((W * L,), jnp.int32),
        pltpu.VMEM((L,), jnp.float32),
    ],
)
def _final(h3m, out_hbm, partv, cv, outv):
    sid = lax.axis_index("s")
    core = lax.axis_index("c")
    iota = lax.iota(jnp.int32, L)
    nb3 = NBS[2]

    @pl.when(jnp.logical_and(sid == 0, core == 0))
    def _():
        soff = 2 * nb3 + W * L
        pltpu.sync_copy(h3m.at[pl.ds(soff, 2 * L)], cv.at[pl.ds(0, 2 * L)])
        prev_prefix = cv[pl.ds(0, L)]
        prev_r = cv[pl.ds(L, L)]
        key_a, r_rem, ceq = _step(h3m, partv, 2, prev_prefix, prev_r)
        bin3 = key_a & (nb3 - 1)

        def nzbody(c, nxt):
            acc = partv[pl.ds(c * L, L)] + partv[pl.ds(nb3 + c * L, L)]
            gi = c * L + iota
            cand = jnp.where(jnp.logical_and(acc > 0, gi > bin3),
                             gi, _full(nb3))
            return jnp.minimum(nxt, cand)

        nxt = lax.fori_loop(0, nb3 // L, nzbody, _full(nb3))
        nxt_bin = jnp.min(nxt)
        keyc_cand = jnp.where(
            nxt_bin < nb3,
            (((key_a >> 10) << 10) | nxt_bin) ^ IMIN, IMAX)

        pltpu.sync_copy(h3m.at[pl.ds(2 * nb3, W * L)], cv)

        def mbody(t, m_acc):
            return jnp.minimum(m_acc, cv[pl.ds(t * L, L)])

        mn_next = jnp.min(lax.fori_loop(
            0, W, mbody, jnp.full((L,), IMAX, jnp.int32)))

        cnt_le = (_full(RANK) - r_rem) + ceq
        keyc_b = jnp.where(cnt_le >= np.int32(RANK + 2),
                           key_a ^ IMIN,
                           jnp.minimum(keyc_cand, mn_next))
        key_b = keyc_b ^ IMIN
        keys = jnp.where(iota == 1, key_b, key_a)
        bits = jnp.where(keys < 0, keys ^ IMIN, ~keys)
        outv[...] = lax.bitcast_convert_type(bits, jnp.float32)
        pltpu.sync_copy(outv, out_hbm)


def kernel(x):
    h1 = _pass1(x)
    h2 = _pass2(x, h1)
    h3 = _pass3(x, h2)
    out = _final(h3)
    return out[0], out[1], out[2]


# speedup vs baseline: 71.2964x; 1.0031x over previous
import functools

import jax
import numpy as np
import jax.numpy as jnp
from jax import lax
from jax.experimental import pallas as pl
from jax.experimental.pallas import tpu as pltpu
from jax.experimental.pallas import tpu_sc as plsc

ROWS, COLS = 64, 32768
N = ROWS * COLS
RANK = N // 2 - 1
L = 16
NC, NS = 2, 16
W = NC * NS
RPW = ROWS // W
CHUNK = N // W
VECS = CHUNK // L
SHIFTS = (21, 10, 0)
NBS = (2048, 2048, 1024)
IMIN = np.int32(-(2 ** 31))
IMAX = np.int32(2 ** 31 - 1)


def _mesh():
    return plsc.VectorSubcoreMesh(core_axis_name="c", subcore_axis_name="s")


def _full(v):
    return jnp.full((L,), v, jnp.int32)


def _key16(v_f32):
    b = lax.bitcast_convert_type(v_f32, jnp.int32)
    m = lax.shift_right_arithmetic(b, _full(31))
    return b ^ (m | IMIN)


def _merge_find(partv, nb, r_rem):
    iota = lax.iota(jnp.int32, L)

    def body(c, carry):
        bin_, below, total, ceq = carry
        acc = partv[pl.ds(c * L, L)] + partv[pl.ds(nb + c * L, L)]
        cs = plsc.cumsum(acc)
        mask = (total + cs) > r_rem
        found_here = jnp.any(mask)
        ffs = jnp.where(found_here, plsc.all_reduce_ffs(mask), np.int32(L))
        already = bin_ >= 0
        new_here = jnp.logical_and(found_here, jnp.logical_not(already))
        bin_ = jnp.where(new_here, c * L + ffs, bin_)
        below_here = jnp.sum(jnp.where(iota < ffs, acc, 0))
        eq_here = jnp.sum(jnp.where(iota == ffs, acc, 0))
        upd = jnp.logical_not(already)
        below = jnp.where(upd, below + below_here, below)
        ceq = jnp.where(jnp.logical_and(upd, found_here), eq_here, ceq)
        total = total + jnp.sum(acc)
        return bin_, below, total, ceq

    bin_, below, _, ceq = lax.fori_loop(
        0, nb // L, body, (_full(-1), _full(0), _full(0), _full(0)))
    return bin_, below, ceq


def _step(h_hbm, partv, p, prefix, r_rem):
    nb = NBS[p]
    pltpu.sync_copy(h_hbm.at[pl.ds(0, 2 * nb)], partv.at[pl.ds(0, 2 * nb)])
    b_, below, ceq = _merge_find(partv, nb, r_rem)
    prefix = (prefix << int(np.log2(nb))) | b_
    return prefix, r_rem - below, ceq


def _make_pass(p):
    nb = NBS[p]
    shift = SHIFTS[p]
    out_words = 2 * nb + (W * L if p == 2 else 0) + (2 * L if p else 0)
    state_off = 2 * nb + (W * L if p == 2 else 0)
    scratch = [
        pltpu.VMEM((CHUNK,), jnp.float32),
        pltpu.VMEM((nb,), jnp.int32),
        pltpu.VMEM((nb,), jnp.int32),
        pltpu.VMEM_SHARED((nb,), jnp.int32),
        pltpu.SemaphoreType.DMA,
    ]
    if p:
        scratch.append(pltpu.VMEM((2 * NBS[p - 1],), jnp.int32))

    @functools.partial(
        pl.kernel,
        out_type=jax.ShapeDtypeStruct((out_words,), jnp.int32),
        mesh=_mesh(),
        compiler_params=pltpu.CompilerParams(needs_layout_passes=False),
        scratch_types=scratch,
    )
    def body(*args):
        x_hbm = args[0]
        if p:
            h_prev = args[1]
            out_hbm = args[2]
            xv, histv, idxv, shared, sem, partv = args[3:]
        else:
            out_hbm = args[1]
            xv, histv, idxv, shared, sem = args[2:]
            partv = None
        sid = lax.axis_index("s")
        core = lax.axis_index("c")
        wid = sid * NC + core
        cps = [
            pltpu.async_copy(
                x_hbm.at[wid * RPW + r], xv.at[pl.ds(r * COLS, COLS)], sem)
            for r in range(RPW)
        ]

        iota = lax.iota(jnp.int32, L)
        zero = jnp.zeros((L,), jnp.int32)

        @plsc.parallel_loop(0, nb // L, unroll=8)
        def _(c):
            histv[pl.ds(c * L, L)] = zero
            idxv[pl.ds(c * L, L)] = iota + c * L

        @pl.when(sid == 0)
        def _():
            pltpu.sync_copy(histv, shared)
        plsc.subcore_barrier()

        if p == 1:
            prefix, r_rem, _ = _step(h_prev, partv, 0, _full(0), _full(RANK))
        elif p == 2:
            poff = 2 * NBS[p - 1]
            pltpu.sync_copy(h_prev.at[pl.ds(poff, 2 * L)],
                            partv.at[pl.ds(0, 2 * L)])
            prev_prefix = partv[pl.ds(0, L)]
            prev_r = partv[pl.ds(L, L)]
            prefix, r_rem, _ = _step(h_prev, partv, 1, prev_prefix, prev_r)
        if p:
            @pl.when(jnp.logical_and(sid == 0, core == 0))
            def _():
                partv[pl.ds(0, L)] = prefix
                partv[pl.ds(L, L)] = r_rem
                pltpu.sync_copy(partv.at[pl.ds(0, 2 * L)],
                                out_hbm.at[pl.ds(state_off, 2 * L)])
        for cp in cps:
            cp.wait()

        ones = jnp.ones((L,), jnp.int32)
        shift_v = _full(shift)
        hi_v = _full(shift + int(np.log2(nb)))
        mask_dig = _full(nb - 1)

        @plsc.parallel_loop(0, VECS, carry=jnp.full((L,), IMAX, jnp.int32),
                            unroll=16)
        def mn(i, mn):
            key = _key16(xv[pl.ds(i * L, L)])
            if shift:
                dig = lax.shift_right_logical(key, shift_v) & mask_dig
            else:
                dig = key & mask_dig
            if p:
                hi = lax.shift_right_logical(key, hi_v)
                m = hi == prefix
                plsc.addupdate_scatter(histv, [dig], ones, mask=m)
                if p == 2:
                    mn = jnp.where(hi > prefix,
                                   jnp.minimum(mn, key ^ IMIN), mn)
            else:
                plsc.addupdate_scatter(histv, [dig], ones)
            return mn

        pltpu.sync_copy(histv, shared.at[idxv], add=True)
        plsc.subcore_barrier()

        @pl.when(sid == 0)
        def _():
            pltpu.sync_copy(shared, out_hbm.at[pl.ds(core * nb, nb)])

        if p == 2:
            idxv[pl.ds(0, L)] = mn
            pltpu.sync_copy(
                idxv.at[pl.ds(0, L)],
                out_hbm.at[pl.ds(2 * nb + wid * L, L)])

    return body


_pass1 = _make_pass(0)
_pass2 = _make_pass(1)
_pass3 = _make_pass(2)


@functools.partial(
    pl.kernel,
    out_type=jax.ShapeDtypeStruct((L,), jnp.float32),
    mesh=_mesh(),
    compiler_params=pltpu.CompilerParams(needs_layout_passes=False),
    scratch_types=[
        pltpu.VMEM((2 * NBS[2],), jnp.int32),
        pltpu.VMEM((W * L,), jnp.int32),
        pltpu.VMEM((L,), jnp.float32),
    ],
)
def _final(h3m, out_hbm, partv, cv, outv):
    sid = lax.axis_index("s")
    core = lax.axis_index("c")
    iota = lax.iota(jnp.int32, L)
    nb3 = NBS[2]

    @pl.when(jnp.logical_and(sid == 0, core == 0))
    def _():
        soff = 2 * nb3 + W * L
        pltpu.sync_copy(h3m.at[pl.ds(soff, 2 * L)], cv.at[pl.ds(0, 2 * L)])
        prev_prefix = cv[pl.ds(0, L)]
        prev_r = cv[pl.ds(L, L)]
        key_a, r_rem, ceq = _step(h3m, partv, 2, prev_prefix, prev_r)
        bin3 = key_a & (nb3 - 1)

        def nzbody(c, nxt):
            acc = partv[pl.ds(c * L, L)] + partv[pl.ds(nb3 + c * L, L)]
            gi = c * L + iota
            cand = jnp.where(jnp.logical_and(acc > 0, gi > bin3),
                             gi, _full(nb3))
            return jnp.minimum(nxt, cand)

        nxt = lax.fori_loop(0, nb3 // L, nzbody, _full(nb3))
        nxt_bin = jnp.min(nxt)
        keyc_cand = jnp.where(
            nxt_bin < nb3,
            (((key_a >> 10) << 10) | nxt_bin) ^ IMIN, IMAX)

        pltpu.sync_copy(h3m.at[pl.ds(2 * nb3, W * L)], cv)

        def mbody(t, m_acc):
            return jnp.minimum(m_acc, cv[pl.ds(t * L, L)])

        mn_next = jnp.min(lax.fori_loop(
            0, W, mbody, jnp.full((L,), IMAX, jnp.int32)))

        cnt_le = (_full(RANK) - r_rem) + ceq
        keyc_b = jnp.where(cnt_le >= np.int32(RANK + 2),
                           key_a ^ IMIN,
                           jnp.minimum(keyc_cand, mn_next))
        key_b = keyc_b ^ IMIN
        keys = jnp.where(iota == 1, key_b, key_a)
        bits = jnp.where(keys < 0, keys ^ IMIN, ~keys)
        outv[...] = lax.bitcast_convert_type(bits, jnp.float32)
        pltpu.sync_copy(outv, out_hbm)


def kernel(x):
    h1 = _pass1(x)
    h2 = _pass2(x, h1)
    h3 = _pass3(x, h2)
    out = _final(h3)
    return out[0], out[1], out[2]
